# Initial kernel scaffold; baseline (speedup 1.0000x reference)
import jax
import jax.numpy as jnp
from jax.experimental import pallas as pl


def kernel(x, edge_index, W_l1, b_l1, W_r1, W_l2, b_l2, W_r2):
    raise NotImplementedError("write your pallas kernel here")


# speedup vs baseline: 3.1506x; 3.1506x over previous
import functools

import jax
import jax.numpy as jnp
from jax import lax
from jax.experimental import pallas as pl
from jax.experimental.pallas import tpu as pltpu, tpu_sc as plsc

N = 10000
E = 320000
D = 128
HD = D // 2

NC = 2
NS = 16
NW = NC * NS
CHUNK = 80
NCHUNK = E // CHUNK
ITERS = NCHUNK // NS
PAD_N = 10240
ROWS_PER_TILE = PAD_N // NS


def _sc_body(tbl, src_h, dst_h, zrows, zcnt, ones_h,
             sums_out, cnt_out, src_v, dst_v, rows_v, stage_v, sem, acc,
             ones_v, cstage_v, cacc):
  c = lax.axis_index("c")
  s = lax.axis_index("s")
  row0 = s * ROWS_PER_TILE

  pltpu.sync_copy(zrows, stage_v)
  pltpu.sync_copy(stage_v, acc.at[pl.ds(row0, ROWS_PER_TILE)])

  @pl.when(c == 0)
  def _():
    pltpu.sync_copy(zcnt, cstage_v)
    pltpu.sync_copy(cstage_v, cacc.at[pl.ds(row0, ROWS_PER_TILE)])

  pltpu.sync_copy(ones_h, ones_v)
  plsc.subcore_barrier()

  def body(i, carry):
    off = (s * ITERS + i) * CHUNK
    pltpu.sync_copy(src_h.at[pl.ds(off, CHUNK)], src_v)
    pltpu.sync_copy(dst_h.at[pl.ds(off, CHUNK)], dst_v)
    for j in range(CHUNK // 16):
      sl = src_v[pl.ds(j * 16, 16)]
      src_v[pl.ds(j * 16, 16)] = sl * 2 + c
    pltpu.async_copy(tbl.at[src_v], rows_v, sem).wait()
    pltpu.sync_copy(rows_v, acc.at[dst_v], add=True)

    @pl.when(c == 0)
    def _():
      pltpu.sync_copy(ones_v, cacc.at[dst_v], add=True)

    return carry

  lax.fori_loop(0, ITERS, body, 0)
  plsc.subcore_barrier()

  pltpu.sync_copy(acc.at[pl.ds(row0, ROWS_PER_TILE)], stage_v)
  pltpu.sync_copy(stage_v,
                  sums_out.at[pl.ds(c * PAD_N + row0, ROWS_PER_TILE)])

  @pl.when(c == 0)
  def _():
    pltpu.sync_copy(cacc.at[pl.ds(row0, ROWS_PER_TILE)], cstage_v)
    pltpu.sync_copy(cstage_v, cnt_out.at[pl.ds(row0, ROWS_PER_TILE)])


_sc_aggregate = pl.kernel(
    _sc_body,
    out_type=(
        jax.ShapeDtypeStruct((NC * PAD_N, HD), jnp.float32),
        jax.ShapeDtypeStruct((PAD_N, 16), jnp.float32),
    ),
    mesh=plsc.VectorSubcoreMesh(core_axis_name="c", subcore_axis_name="s",
                                num_cores=NC, num_subcores=NS),
    compiler_params=pltpu.CompilerParams(use_tc_tiling_on_sc=False),
    scratch_types=[
        pltpu.VMEM((CHUNK,), jnp.int32),
        pltpu.VMEM((CHUNK,), jnp.int32),
        pltpu.VMEM((CHUNK, HD), jnp.float32),
        pltpu.VMEM((ROWS_PER_TILE, HD), jnp.float32),
        pltpu.SemaphoreType.DMA,
        pltpu.VMEM_SHARED((PAD_N, HD), jnp.float32),
        pltpu.VMEM((CHUNK, 16), jnp.float32),
        pltpu.VMEM((ROWS_PER_TILE, 16), jnp.float32),
        pltpu.VMEM_SHARED((PAD_N, 16), jnp.float32),
    ],
)


def _combine_body(relu, sums_ref, cnt_ref, x_ref, wl_ref, bl_ref, wr_ref,
                  o_ref):
  ssum = jnp.concatenate([sums_ref[0], sums_ref[1]], axis=1)
  cnt = cnt_ref[:, 0:1]
  mean = ssum / jnp.maximum(cnt, 1.0)
  o = (jnp.dot(mean, wl_ref[...], preferred_element_type=jnp.float32)
       + bl_ref[...]
       + jnp.dot(x_ref[...], wr_ref[...], preferred_element_type=jnp.float32))
  if relu:
    o = jnp.maximum(o, 0.0)
  o_ref[...] = o


def _tc_combine(sums, cnt, x, wl_t, bl, wr_t, relu):
  B = 1000
  grid = N // B
  return pl.pallas_call(
      functools.partial(_combine_body, relu),
      grid=(grid,),
      in_specs=[
          pl.BlockSpec((NC, B, HD), lambda i: (0, i, 0)),
          pl.BlockSpec((B, 16), lambda i: (i, 0)),
          pl.BlockSpec((B, D), lambda i: (i, 0)),
          pl.BlockSpec((D, D), lambda i: (0, 0)),
          pl.BlockSpec((1, D), lambda i: (0, 0)),
          pl.BlockSpec((D, D), lambda i: (0, 0)),
      ],
      out_specs=pl.BlockSpec((B, D), lambda i: (i, 0)),
      out_shape=jax.ShapeDtypeStruct((N, D), jnp.float32),
  )(sums, cnt, x, wl_t, bl, wr_t)


def kernel(x, edge_index, W_l1, b_l1, W_r1, W_l2, b_l2, W_r2):
  src = edge_index[0].astype(jnp.int32)
  dst = edge_index[1].astype(jnp.int32)

  zrows = jnp.zeros((ROWS_PER_TILE, HD), jnp.float32)
  zcnt = jnp.zeros((ROWS_PER_TILE, 16), jnp.float32)
  ones = jnp.ones((CHUNK, 16), jnp.float32)

  sums1, cnt = _sc_aggregate(x.reshape(2 * N, HD), src, dst,
                             zrows, zcnt, ones)
  sums1 = sums1.reshape(NC, PAD_N, HD)

  h = _tc_combine(sums1, cnt, x, W_l1.T, b_l1.reshape(1, D), W_r1.T,
                  relu=True)

  sums2, _ = _sc_aggregate(h.reshape(2 * N, HD), src, dst,
                           zrows, zcnt, ones)
  sums2 = sums2.reshape(NC, PAD_N, HD)
  out = _tc_combine(sums2, cnt, h, W_l2.T, b_l2.reshape(1, D), W_r2.T,
                    relu=False)
  return out


# speedup vs baseline: 7.2241x; 2.2929x over previous
import functools

import jax
import jax.numpy as jnp
from jax import lax
from jax.experimental import pallas as pl
from jax.experimental.pallas import tpu as pltpu, tpu_sc as plsc

N = 10000
E = 320000
D = 128
HD = D // 2

NC = 2
NS = 16
NW = NC * NS
CHUNK = 80
NCHUNK = E // CHUNK
ITERS = NCHUNK // NS
NBUF = 5
PAD_N = 10240
ROWS_PER_TILE = PAD_N // NS


def _sc_body(tbl, src_h, dst_h, zrows, zcnt, ones_h,
             sums_out, cnt_out, src_v, dst_v, rows_v, stage_v,
             sem_i, sem_g, acc, ones_v, cstage_v, cacc):
  c = lax.axis_index("c")
  s = lax.axis_index("s")
  row0 = s * ROWS_PER_TILE

  pltpu.sync_copy(zrows, stage_v)
  pltpu.sync_copy(stage_v, acc.at[pl.ds(row0, ROWS_PER_TILE)])

  @pl.when(c == 0)
  def _():
    pltpu.sync_copy(zcnt, cstage_v)
    pltpu.sync_copy(cstage_v, cacc.at[pl.ds(row0, ROWS_PER_TILE)])

  pltpu.sync_copy(ones_h, ones_v)
  plsc.subcore_barrier()

  def body(i2, carry):
    idx_waits = []
    for k in range(NBUF):
      off = (s * ITERS + i2 * NBUF + k) * CHUNK
      w1 = pltpu.async_copy(src_h.at[pl.ds(off, CHUNK)], src_v[k], sem_i[k])
      w2 = pltpu.async_copy(dst_h.at[pl.ds(off, CHUNK)], dst_v[k], sem_i[k])
      idx_waits.append((w1, w2))
    gather_waits = []
    for k in range(NBUF):
      w1, w2 = idx_waits[k]
      w1.wait()
      w2.wait()
      for j in range(CHUNK // 16):
        sl = src_v[k][pl.ds(j * 16, 16)]
        src_v[k][pl.ds(j * 16, 16)] = sl * 2 + c
      gather_waits.append(
          pltpu.async_copy(tbl.at[src_v[k]], rows_v[k], sem_g[k]))
    for k in range(NBUF):
      gather_waits[k].wait()
      pltpu.sync_copy(rows_v[k], acc.at[dst_v[k]], add=True)

      @pl.when(c == 0)
      def _(k=k):
        pltpu.sync_copy(ones_v, cacc.at[dst_v[k]], add=True)

    return carry

  lax.fori_loop(0, ITERS // NBUF, body, 0)
  plsc.subcore_barrier()

  pltpu.sync_copy(acc.at[pl.ds(row0, ROWS_PER_TILE)], stage_v)
  pltpu.sync_copy(stage_v,
                  sums_out.at[pl.ds(c * PAD_N + row0, ROWS_PER_TILE)])

  @pl.when(c == 0)
  def _():
    pltpu.sync_copy(cacc.at[pl.ds(row0, ROWS_PER_TILE)], cstage_v)
    pltpu.sync_copy(cstage_v, cnt_out.at[pl.ds(row0, ROWS_PER_TILE)])


_sc_aggregate = pl.kernel(
    _sc_body,
    out_type=(
        jax.ShapeDtypeStruct((NC * PAD_N, HD), jnp.float32),
        jax.ShapeDtypeStruct((PAD_N, 16), jnp.float32),
    ),
    mesh=plsc.VectorSubcoreMesh(core_axis_name="c", subcore_axis_name="s",
                                num_cores=NC, num_subcores=NS),
    compiler_params=pltpu.CompilerParams(use_tc_tiling_on_sc=False),
    scratch_types=[
        [pltpu.VMEM((CHUNK,), jnp.int32) for _ in range(NBUF)],
        [pltpu.VMEM((CHUNK,), jnp.int32) for _ in range(NBUF)],
        [pltpu.VMEM((CHUNK, HD), jnp.float32) for _ in range(NBUF)],
        pltpu.VMEM((ROWS_PER_TILE, HD), jnp.float32),
        [pltpu.SemaphoreType.DMA for _ in range(NBUF)],
        [pltpu.SemaphoreType.DMA for _ in range(NBUF)],
        pltpu.VMEM_SHARED((PAD_N, HD), jnp.float32),
        pltpu.VMEM((CHUNK, 16), jnp.float32),
        pltpu.VMEM((ROWS_PER_TILE, 16), jnp.float32),
        pltpu.VMEM_SHARED((PAD_N, 16), jnp.float32),
    ],
)


def _combine_body(relu, sums_ref, cnt_ref, x_ref, wl_ref, bl_ref, wr_ref,
                  o_ref):
  ssum = jnp.concatenate([sums_ref[0], sums_ref[1]], axis=1)
  cnt = cnt_ref[:, 0:1]
  mean = ssum / jnp.maximum(cnt, 1.0)
  o = (jnp.dot(mean, wl_ref[...], preferred_element_type=jnp.float32)
       + bl_ref[...]
       + jnp.dot(x_ref[...], wr_ref[...], preferred_element_type=jnp.float32))
  if relu:
    o = jnp.maximum(o, 0.0)
  o_ref[...] = o


def _tc_combine(sums, cnt, x, wl_t, bl, wr_t, relu):
  B = 1000
  grid = N // B
  return pl.pallas_call(
      functools.partial(_combine_body, relu),
      grid=(grid,),
      in_specs=[
          pl.BlockSpec((NC, B, HD), lambda i: (0, i, 0)),
          pl.BlockSpec((B, 16), lambda i: (i, 0)),
          pl.BlockSpec((B, D), lambda i: (i, 0)),
          pl.BlockSpec((D, D), lambda i: (0, 0)),
          pl.BlockSpec((1, D), lambda i: (0, 0)),
          pl.BlockSpec((D, D), lambda i: (0, 0)),
      ],
      out_specs=pl.BlockSpec((B, D), lambda i: (i, 0)),
      out_shape=jax.ShapeDtypeStruct((N, D), jnp.float32),
  )(sums, cnt, x, wl_t, bl, wr_t)


def kernel(x, edge_index, W_l1, b_l1, W_r1, W_l2, b_l2, W_r2):
  src = edge_index[0].astype(jnp.int32)
  dst = edge_index[1].astype(jnp.int32)

  zrows = jnp.zeros((ROWS_PER_TILE, HD), jnp.float32)
  zcnt = jnp.zeros((ROWS_PER_TILE, 16), jnp.float32)
  ones = jnp.ones((CHUNK, 16), jnp.float32)

  sums1, cnt = _sc_aggregate(x.reshape(2 * N, HD), src, dst,
                             zrows, zcnt, ones)
  sums1 = sums1.reshape(NC, PAD_N, HD)

  h = _tc_combine(sums1, cnt, x, W_l1.T, b_l1.reshape(1, D), W_r1.T,
                  relu=True)

  sums2, _ = _sc_aggregate(h.reshape(2 * N, HD), src, dst,
                           zrows, zcnt, ones)
  sums2 = sums2.reshape(NC, PAD_N, HD)
  out = _tc_combine(sums2, cnt, h, W_l2.T, b_l2.reshape(1, D), W_r2.T,
                    relu=False)
  return out


# speedup vs baseline: 8.1673x; 1.1306x over previous
import functools

import jax
import jax.numpy as jnp
from jax import lax
from jax.experimental import pallas as pl
from jax.experimental.pallas import tpu as pltpu, tpu_sc as plsc

N = 10000
E = 320000
D = 128
HD = D // 2

NC = 2
NS = 16
NW = NC * NS
CHUNK = 80
NCHUNK = E // CHUNK
ITERS = NCHUNK // NS
NBUF = 5
PAD_N = 10240
ROWS_PER_TILE = PAD_N // NS


def _sc_body(with_counts, *refs):
  if with_counts:
    (tbl, src_h, dst_h, zrows, zcnt, ones_h,
     sums_out, cnt_out, src_v, dst_v, rows_v, stage_v,
     sem_ia, sem_ib, sem_g, sem_s, sem_o,
     acc, ones_v, cstage_v, cacc) = refs
  else:
    (tbl, src_h, dst_h, zrows,
     sums_out, src_v, dst_v, rows_v, stage_v,
     sem_ia, sem_ib, sem_g, sem_s,
     acc) = refs
  c = lax.axis_index("c")
  s = lax.axis_index("s")
  row0 = s * ROWS_PER_TILE

  pltpu.sync_copy(zrows, stage_v)
  pltpu.sync_copy(stage_v, acc.at[pl.ds(row0, ROWS_PER_TILE)])
  if with_counts:
    pltpu.sync_copy(zcnt, cstage_v)
    pltpu.sync_copy(cstage_v, cacc.at[pl.ds(row0, ROWS_PER_TILE)])
    pltpu.sync_copy(ones_h, ones_v)
  plsc.subcore_barrier()

  def body(i2, carry):
    chunk0 = s * ITERS + i2 * NBUF
    wi_s = pltpu.async_copy(src_h.at[pl.ds(chunk0, NBUF)], src_v, sem_ia)
    wi_d = pltpu.async_copy(dst_h.at[pl.ds(chunk0, NBUF)], dst_v, sem_ib)
    wi_s.wait()
    wi_d.wait()
    gather_waits = []
    for k in range(NBUF):
      for j in range(CHUNK // 16):
        sl = src_v[k, pl.ds(j * 16, 16)]
        src_v[k, pl.ds(j * 16, 16)] = sl * 2 + c
      gather_waits.append(
          pltpu.async_copy(tbl.at[src_v.at[k]], rows_v[k], sem_g[k]))
    scatter_waits = []
    for k in range(NBUF):
      gather_waits[k].wait()
      scatter_waits.append(
          pltpu.async_copy(rows_v[k], acc.at[dst_v.at[k]], sem_s[k],
                           add=True))
      if with_counts:
        @pl.when(c == (k % 2))
        def _(k=k):
          pltpu.async_copy(ones_v, cacc.at[dst_v.at[k]], sem_o[k],
                           add=True).wait()
    for k in range(NBUF):
      scatter_waits[k].wait()
    return carry

  lax.fori_loop(0, ITERS // NBUF, body, 0)
  plsc.subcore_barrier()

  pltpu.sync_copy(acc.at[pl.ds(row0, ROWS_PER_TILE)], stage_v)
  pltpu.sync_copy(stage_v,
                  sums_out.at[pl.ds(c * PAD_N + row0, ROWS_PER_TILE)])
  if with_counts:
    pltpu.sync_copy(cacc.at[pl.ds(row0, ROWS_PER_TILE)], cstage_v)
    pltpu.sync_copy(cstage_v,
                    cnt_out.at[pl.ds(c * PAD_N + row0, ROWS_PER_TILE)])


_MESH = plsc.VectorSubcoreMesh(core_axis_name="c", subcore_axis_name="s",
                               num_cores=NC, num_subcores=NS)

_common_scratch = lambda: [
    pltpu.VMEM((NBUF, CHUNK), jnp.int32),
    pltpu.VMEM((NBUF, CHUNK), jnp.int32),
    [pltpu.VMEM((CHUNK, HD), jnp.float32) for _ in range(NBUF)],
    pltpu.VMEM((ROWS_PER_TILE, HD), jnp.float32),
    pltpu.SemaphoreType.DMA,
    pltpu.SemaphoreType.DMA,
    [pltpu.SemaphoreType.DMA for _ in range(NBUF)],
    [pltpu.SemaphoreType.DMA for _ in range(NBUF)],
]

_sc_aggregate_counts = pl.kernel(
    functools.partial(_sc_body, True),
    out_type=(
        jax.ShapeDtypeStruct((NC * PAD_N, HD), jnp.float32),
        jax.ShapeDtypeStruct((NC * PAD_N, 16), jnp.float32),
    ),
    mesh=_MESH,
    compiler_params=pltpu.CompilerParams(use_tc_tiling_on_sc=False),
    scratch_types=_common_scratch() + [
        [pltpu.SemaphoreType.DMA for _ in range(NBUF)],
        pltpu.VMEM_SHARED((PAD_N, HD), jnp.float32),
        pltpu.VMEM((CHUNK, 16), jnp.float32),
        pltpu.VMEM((ROWS_PER_TILE, 16), jnp.float32),
        pltpu.VMEM_SHARED((PAD_N, 16), jnp.float32),
    ],
)

_sc_aggregate_plain = pl.kernel(
    functools.partial(_sc_body, False),
    out_type=jax.ShapeDtypeStruct((NC * PAD_N, HD), jnp.float32),
    mesh=_MESH,
    compiler_params=pltpu.CompilerParams(use_tc_tiling_on_sc=False),
    scratch_types=_common_scratch() + [
        pltpu.VMEM_SHARED((PAD_N, HD), jnp.float32),
    ],
)


def _combine_body(relu, sums_ref, cnt_ref, x_ref, wl_ref, bl_ref, wr_ref,
                  o_ref):
  ssum = jnp.concatenate([sums_ref[0], sums_ref[1]], axis=1)
  cnt = cnt_ref[0, :, 0:1] + cnt_ref[1, :, 0:1]
  mean = ssum / jnp.maximum(cnt, 1.0)
  o = (jnp.dot(mean, wl_ref[...], preferred_element_type=jnp.float32)
       + bl_ref[...]
       + jnp.dot(x_ref[...], wr_ref[...], preferred_element_type=jnp.float32))
  if relu:
    o = jnp.maximum(o, 0.0)
  o_ref[...] = o


def _tc_combine(sums, cnt, x, wl_t, bl, wr_t, relu):
  B = 1000
  grid = N // B
  return pl.pallas_call(
      functools.partial(_combine_body, relu),
      grid=(grid,),
      in_specs=[
          pl.BlockSpec((NC, B, HD), lambda i: (0, i, 0)),
          pl.BlockSpec((NC, B, 16), lambda i: (0, i, 0)),
          pl.BlockSpec((B, D), lambda i: (i, 0)),
          pl.BlockSpec((D, D), lambda i: (0, 0)),
          pl.BlockSpec((1, D), lambda i: (0, 0)),
          pl.BlockSpec((D, D), lambda i: (0, 0)),
      ],
      out_specs=pl.BlockSpec((B, D), lambda i: (i, 0)),
      out_shape=jax.ShapeDtypeStruct((N, D), jnp.float32),
  )(sums, cnt, x, wl_t, bl, wr_t)


def kernel(x, edge_index, W_l1, b_l1, W_r1, W_l2, b_l2, W_r2):
  src = edge_index[0].astype(jnp.int32).reshape(NCHUNK, CHUNK)
  dst = edge_index[1].astype(jnp.int32).reshape(NCHUNK, CHUNK)

  zrows = jnp.zeros((ROWS_PER_TILE, HD), jnp.float32)
  zcnt = jnp.zeros((ROWS_PER_TILE, 16), jnp.float32)
  ones = jnp.ones((CHUNK, 16), jnp.float32)

  sums1, cnt = _sc_aggregate_counts(x.reshape(2 * N, HD), src, dst,
                                    zrows, zcnt, ones)
  sums1 = sums1.reshape(NC, PAD_N, HD)
  cnt = cnt.reshape(NC, PAD_N, 16)

  h = _tc_combine(sums1, cnt, x, W_l1.T, b_l1.reshape(1, D), W_r1.T,
                  relu=True)

  sums2 = _sc_aggregate_plain(h.reshape(2 * N, HD), src, dst, zrows)
  sums2 = sums2.reshape(NC, PAD_N, HD)
  out = _tc_combine(sums2, cnt, h, W_l2.T, b_l2.reshape(1, D), W_r2.T,
                    relu=False)
  return out


# speedup vs baseline: 9.3444x; 1.1441x over previous
import functools

import jax
import jax.numpy as jnp
from jax import lax
from jax.experimental import pallas as pl
from jax.experimental.pallas import tpu as pltpu, tpu_sc as plsc

N = 10000
E = 320000
D = 128
HD = D // 2

NC = 2
NS = 16
NW = NC * NS
CHUNK = 80
NCHUNK = E // CHUNK
ITERS = NCHUNK // NS
NBUF = 5
PAD_N = 10240
ROWS_PER_TILE = PAD_N // NS
STAGE_ROWS = 128
NSTAGE = ROWS_PER_TILE // STAGE_ROWS


def _sc_body(with_counts, *refs):
  if with_counts:
    (tbl, src_h, dst_h, zrows, zcnt, ones_h,
     sums_out, cnt_out,
     src_a, dst_a, src_b, dst_b, rows_v, stage_v,
     sem_i, sem_g, sem_s, sem_o,
     acc, ones_v, cstage_v, cacc) = refs
  else:
    (tbl, src_h, dst_h, zrows,
     sums_out,
     src_a, dst_a, src_b, dst_b, rows_v, stage_v,
     sem_i, sem_g, sem_s,
     acc) = refs
    sem_o = None
  c = lax.axis_index("c")
  s = lax.axis_index("s")
  row0 = s * ROWS_PER_TILE

  pltpu.sync_copy(zrows, stage_v)
  for t in range(NSTAGE):
    pltpu.sync_copy(stage_v, acc.at[pl.ds(row0 + t * STAGE_ROWS, STAGE_ROWS)])
  if with_counts:
    pltpu.sync_copy(zcnt, cstage_v)
    for t in range(NSTAGE):
      pltpu.sync_copy(cstage_v,
                      cacc.at[pl.ds(row0 + t * STAGE_ROWS, STAGE_ROWS)])
    pltpu.sync_copy(ones_h, ones_v)
  plsc.subcore_barrier()

  base = s * ITERS
  nbody = ITERS // NBUF

  def fire_idx(chunk0, sv, dv, s_sem, d_sem):
    pltpu.async_copy(src_h.at[pl.ds(chunk0, NBUF)], sv, s_sem)
    pltpu.async_copy(dst_h.at[pl.ds(chunk0, NBUF)], dv, d_sem)

  def wait_idx(chunk0, sv, dv, s_sem, d_sem):
    pltpu.make_async_copy(src_h.at[pl.ds(chunk0, NBUF)], sv, s_sem).wait()
    pltpu.make_async_copy(dst_h.at[pl.ds(chunk0, NBUF)], dv, d_sem).wait()

  def subbody(chunk0, sv, dv, s_sem, d_sem, prefetch):
    prefetch()
    wait_idx(chunk0, sv, dv, s_sem, d_sem)
    gather_waits = []
    for k in range(NBUF):
      for j in range(CHUNK // 16):
        sl = sv[k, pl.ds(j * 16, 16)]
        sv[k, pl.ds(j * 16, 16)] = sl * 2 + c
      gather_waits.append(
          pltpu.async_copy(tbl.at[sv.at[k]], rows_v[k], sem_g[k]))
    scatter_waits = []
    for k in range(NBUF):
      gather_waits[k].wait()
      scatter_waits.append(
          pltpu.async_copy(rows_v[k], acc.at[dv.at[k]], sem_s[k], add=True))
      if with_counts:
        scatter_waits.append(
            pltpu.async_copy(ones_v, cacc.at[dv.at[k]], sem_o[k], add=True))
    for w in scatter_waits:
      w.wait()

  fire_idx(base, src_a, dst_a, sem_i[0], sem_i[1])

  def body(i2, carry):
    c0 = base + (2 * i2) * NBUF

    def prefetch_b():
      fire_idx(c0 + NBUF, src_b, dst_b, sem_i[2], sem_i[3])

    def prefetch_a():
      @pl.when(i2 < nbody // 2 - 1)
      def _():
        fire_idx(c0 + 2 * NBUF, src_a, dst_a, sem_i[0], sem_i[1])

    subbody(c0, src_a, dst_a, sem_i[0], sem_i[1], prefetch_b)
    subbody(c0 + NBUF, src_b, dst_b, sem_i[2], sem_i[3], prefetch_a)
    return carry

  lax.fori_loop(0, nbody // 2, body, 0)
  plsc.subcore_barrier()

  for t in range(NSTAGE):
    pltpu.sync_copy(acc.at[pl.ds(row0 + t * STAGE_ROWS, STAGE_ROWS)], stage_v)
    pltpu.sync_copy(
        stage_v,
        sums_out.at[pl.ds(c * PAD_N + row0 + t * STAGE_ROWS, STAGE_ROWS)])
  if with_counts:
    for t in range(NSTAGE):
      pltpu.sync_copy(cacc.at[pl.ds(row0 + t * STAGE_ROWS, STAGE_ROWS)],
                      cstage_v)
      pltpu.sync_copy(
          cstage_v,
          cnt_out.at[pl.ds(c * PAD_N + row0 + t * STAGE_ROWS, STAGE_ROWS)])


_MESH = plsc.VectorSubcoreMesh(core_axis_name="c", subcore_axis_name="s",
                               num_cores=NC, num_subcores=NS)

_common_scratch = lambda: [
    pltpu.VMEM((NBUF, CHUNK), jnp.int32),
    pltpu.VMEM((NBUF, CHUNK), jnp.int32),
    pltpu.VMEM((NBUF, CHUNK), jnp.int32),
    pltpu.VMEM((NBUF, CHUNK), jnp.int32),
    [pltpu.VMEM((CHUNK, HD), jnp.float32) for _ in range(NBUF)],
    pltpu.VMEM((STAGE_ROWS, HD), jnp.float32),
    [pltpu.SemaphoreType.DMA for _ in range(4)],
    [pltpu.SemaphoreType.DMA for _ in range(NBUF)],
    [pltpu.SemaphoreType.DMA for _ in range(NBUF)],
]

_sc_aggregate_counts = pl.kernel(
    functools.partial(_sc_body, True),
    out_type=(
        jax.ShapeDtypeStruct((NC * PAD_N, HD), jnp.float32),
        jax.ShapeDtypeStruct((NC * PAD_N, 16), jnp.float32),
    ),
    mesh=_MESH,
    compiler_params=pltpu.CompilerParams(use_tc_tiling_on_sc=False),
    scratch_types=_common_scratch() + [
        [pltpu.SemaphoreType.DMA for _ in range(NBUF)],
        pltpu.VMEM_SHARED((PAD_N, HD), jnp.float32),
        pltpu.VMEM((CHUNK, 16), jnp.float32),
        pltpu.VMEM((STAGE_ROWS, 16), jnp.float32),
        pltpu.VMEM_SHARED((PAD_N, 16), jnp.float32),
    ],
)

_sc_aggregate_plain = pl.kernel(
    functools.partial(_sc_body, False),
    out_type=jax.ShapeDtypeStruct((NC * PAD_N, HD), jnp.float32),
    mesh=_MESH,
    compiler_params=pltpu.CompilerParams(use_tc_tiling_on_sc=False),
    scratch_types=_common_scratch() + [
        pltpu.VMEM_SHARED((PAD_N, HD), jnp.float32),
    ],
)


def _combine_body(relu, sums_ref, cnt_ref, x_ref, wl_ref, bl_ref, wr_ref,
                  o_ref):
  ssum = jnp.concatenate([sums_ref[0], sums_ref[1]], axis=1)
  cnt = (cnt_ref[0, :, 0:1] + cnt_ref[1, :, 0:1]) * 0.5
  mean = ssum / jnp.maximum(cnt, 1.0)
  o = (jnp.dot(mean, wl_ref[...], preferred_element_type=jnp.float32)
       + bl_ref[...]
       + jnp.dot(x_ref[...], wr_ref[...], preferred_element_type=jnp.float32))
  if relu:
    o = jnp.maximum(o, 0.0)
  o_ref[...] = o


def _tc_combine(sums, cnt, x, wl_t, bl, wr_t, relu):
  B = 1000
  grid = N // B
  return pl.pallas_call(
      functools.partial(_combine_body, relu),
      grid=(grid,),
      in_specs=[
          pl.BlockSpec((NC, B, HD), lambda i: (0, i, 0)),
          pl.BlockSpec((NC, B, 16), lambda i: (0, i, 0)),
          pl.BlockSpec((B, D), lambda i: (i, 0)),
          pl.BlockSpec((D, D), lambda i: (0, 0)),
          pl.BlockSpec((1, D), lambda i: (0, 0)),
          pl.BlockSpec((D, D), lambda i: (0, 0)),
      ],
      out_specs=pl.BlockSpec((B, D), lambda i: (i, 0)),
      out_shape=jax.ShapeDtypeStruct((N, D), jnp.float32),
  )(sums, cnt, x, wl_t, bl, wr_t)


def kernel(x, edge_index, W_l1, b_l1, W_r1, W_l2, b_l2, W_r2):
  src = edge_index[0].astype(jnp.int32).reshape(NCHUNK, CHUNK)
  dst = edge_index[1].astype(jnp.int32).reshape(NCHUNK, CHUNK)

  zrows = jnp.zeros((STAGE_ROWS, HD), jnp.float32)
  zcnt = jnp.zeros((STAGE_ROWS, 16), jnp.float32)
  ones = jnp.ones((CHUNK, 16), jnp.float32)

  sums1, cnt = _sc_aggregate_counts(x.reshape(2 * N, HD), src, dst,
                                    zrows, zcnt, ones)
  sums1 = sums1.reshape(NC, PAD_N, HD)
  cnt = cnt.reshape(NC, PAD_N, 16)

  h = _tc_combine(sums1, cnt, x, W_l1.T, b_l1.reshape(1, D), W_r1.T,
                  relu=True)

  sums2 = _sc_aggregate_plain(h.reshape(2 * N, HD), src, dst, zrows)
  sums2 = sums2.reshape(NC, PAD_N, HD)
  out = _tc_combine(sums2, cnt, h, W_l2.T, b_l2.reshape(1, D), W_r2.T,
                    relu=False)
  return out


# speedup vs baseline: 11.5294x; 1.2338x over previous
import functools

import jax
import jax.numpy as jnp
from jax import lax
from jax.experimental import pallas as pl
from jax.experimental.pallas import tpu as pltpu, tpu_sc as plsc

N = 10000
E = 320000
D = 128
HD = D // 2

NC = 2
NS = 16
NW = NC * NS
CHUNK = 80
NCHUNK = E // CHUNK
ITERS = NCHUNK // NS
NBUF = 5
PAD_N = 10240
ROWS_PER_TILE = PAD_N // NS
STAGE_ROWS = 128
NSTAGE = ROWS_PER_TILE // STAGE_ROWS


def _sc_body(with_counts, *refs):
  if with_counts:
    (tbl, src_h, dst_h, zrows, zcnt, ones_h,
     sums_out, cnt_out,
     src_a, dst_a, src_b, dst_b, sdst, rows_v, stage_v,
     sem_i, sem_g, sem_s, sem_o,
     acc, ones_v, cstage_v, cacc) = refs
  else:
    (tbl, src_h, dst_h, zrows,
     sums_out,
     src_a, dst_a, src_b, dst_b, sdst, rows_v, stage_v,
     sem_i, sem_g, sem_s,
     acc) = refs
    sem_o = None
  c = lax.axis_index("c")
  s = lax.axis_index("s")
  row0 = s * ROWS_PER_TILE

  pltpu.sync_copy(zrows, stage_v)
  for t in range(NSTAGE):
    pltpu.sync_copy(stage_v, acc.at[pl.ds(row0 + t * STAGE_ROWS, STAGE_ROWS)])
  if with_counts:
    pltpu.sync_copy(zcnt, cstage_v)
    for t in range(NSTAGE):
      pltpu.sync_copy(cstage_v,
                      cacc.at[pl.ds(row0 + t * STAGE_ROWS, STAGE_ROWS)])
    pltpu.sync_copy(ones_h, ones_v)
  plsc.subcore_barrier()

  base = s * ITERS
  nbody = ITERS // NBUF

  def fire_idx(chunk0, sv, dv, s_sem, d_sem):
    pltpu.async_copy(src_h.at[pl.ds(chunk0, NBUF)], sv, s_sem)
    pltpu.async_copy(dst_h.at[pl.ds(chunk0, NBUF)], dv, d_sem)

  def wait_idx(chunk0, sv, dv, s_sem, d_sem):
    pltpu.make_async_copy(src_h.at[pl.ds(chunk0, NBUF)], sv, s_sem).wait()
    pltpu.make_async_copy(dst_h.at[pl.ds(chunk0, NBUF)], dv, d_sem).wait()

  def drain_scatters(half):
    for k in range(half * NBUF, (half + 1) * NBUF):
      pltpu.make_async_copy(tbl.at[pl.ds(0, CHUNK)], rows_v[k],
                            sem_s[k]).wait()
      if with_counts:
        pltpu.make_async_copy(zcnt.at[pl.ds(0, CHUNK)], ones_v,
                              sem_o[k]).wait()

  def subbody(chunk0, half, sv, dv, s_sem, d_sem, prefetch, first):
    prefetch()
    wait_idx(chunk0, sv, dv, s_sem, d_sem)

    @pl.when(jnp.logical_not(first))
    def _():
      drain_scatters(half)

    gather_waits = []
    for k in range(NBUF):
      for j in range(CHUNK // 16):
        sl = sv[k, pl.ds(j * 16, 16)]
        sv[k, pl.ds(j * 16, 16)] = sl * 2 + c
      gather_waits.append(
          pltpu.async_copy(tbl.at[sv.at[k]], rows_v[half * NBUF + k],
                           sem_g[k]))
    for k in range(NBUF):
      for j in range(CHUNK // 16):
        sdst[half * NBUF + k, pl.ds(j * 16, 16)] = dv[k, pl.ds(j * 16, 16)]
    for k in range(NBUF):
      gather_waits[k].wait()
      pltpu.async_copy(rows_v[half * NBUF + k],
                       acc.at[sdst.at[half * NBUF + k]],
                       sem_s[half * NBUF + k], add=True)
      if with_counts:
        pltpu.async_copy(ones_v, cacc.at[sdst.at[half * NBUF + k]],
                         sem_o[half * NBUF + k], add=True)

  fire_idx(base, src_a, dst_a, sem_i[0], sem_i[1])

  def body(i2, carry):
    c0 = base + (2 * i2) * NBUF

    def prefetch_b():
      fire_idx(c0 + NBUF, src_b, dst_b, sem_i[2], sem_i[3])

    def prefetch_a():
      @pl.when(i2 < nbody // 2 - 1)
      def _():
        fire_idx(c0 + 2 * NBUF, src_a, dst_a, sem_i[0], sem_i[1])

    subbody(c0, 0, src_a, dst_a, sem_i[0], sem_i[1], prefetch_b, i2 == 0)
    subbody(c0 + NBUF, 1, src_b, dst_b, sem_i[2], sem_i[3], prefetch_a,
            i2 == 0)
    return carry

  lax.fori_loop(0, nbody // 2, body, 0)
  drain_scatters(0)
  drain_scatters(1)
  plsc.subcore_barrier()

  for t in range(NSTAGE):
    pltpu.sync_copy(acc.at[pl.ds(row0 + t * STAGE_ROWS, STAGE_ROWS)], stage_v)
    pltpu.sync_copy(
        stage_v,
        sums_out.at[pl.ds(c * PAD_N + row0 + t * STAGE_ROWS, STAGE_ROWS)])
  if with_counts:
    for t in range(NSTAGE):
      pltpu.sync_copy(cacc.at[pl.ds(row0 + t * STAGE_ROWS, STAGE_ROWS)],
                      cstage_v)
      pltpu.sync_copy(
          cstage_v,
          cnt_out.at[pl.ds(c * PAD_N + row0 + t * STAGE_ROWS, STAGE_ROWS)])


_MESH = plsc.VectorSubcoreMesh(core_axis_name="c", subcore_axis_name="s",
                               num_cores=NC, num_subcores=NS)

_common_scratch = lambda: [
    pltpu.VMEM((NBUF, CHUNK), jnp.int32),
    pltpu.VMEM((NBUF, CHUNK), jnp.int32),
    pltpu.VMEM((NBUF, CHUNK), jnp.int32),
    pltpu.VMEM((NBUF, CHUNK), jnp.int32),
    pltpu.VMEM((2 * NBUF, CHUNK), jnp.int32),
    [pltpu.VMEM((CHUNK, HD), jnp.float32) for _ in range(2 * NBUF)],
    pltpu.VMEM((STAGE_ROWS, HD), jnp.float32),
    [pltpu.SemaphoreType.DMA for _ in range(4)],
    [pltpu.SemaphoreType.DMA for _ in range(NBUF)],
    [pltpu.SemaphoreType.DMA for _ in range(2 * NBUF)],
]

_sc_aggregate_counts = pl.kernel(
    functools.partial(_sc_body, True),
    out_type=(
        jax.ShapeDtypeStruct((NC * PAD_N, HD), jnp.float32),
        jax.ShapeDtypeStruct((NC * PAD_N, 16), jnp.float32),
    ),
    mesh=_MESH,
    compiler_params=pltpu.CompilerParams(use_tc_tiling_on_sc=False),
    scratch_types=_common_scratch() + [
        [pltpu.SemaphoreType.DMA for _ in range(2 * NBUF)],
        pltpu.VMEM_SHARED((PAD_N, HD), jnp.float32),
        pltpu.VMEM((CHUNK, 16), jnp.float32),
        pltpu.VMEM((STAGE_ROWS, 16), jnp.float32),
        pltpu.VMEM_SHARED((PAD_N, 16), jnp.float32),
    ],
)

_sc_aggregate_plain = pl.kernel(
    functools.partial(_sc_body, False),
    out_type=jax.ShapeDtypeStruct((NC * PAD_N, HD), jnp.float32),
    mesh=_MESH,
    compiler_params=pltpu.CompilerParams(use_tc_tiling_on_sc=False),
    scratch_types=_common_scratch() + [
        pltpu.VMEM_SHARED((PAD_N, HD), jnp.float32),
    ],
)


def _combine_body(relu, sums_ref, cnt_ref, x_ref, wl_ref, bl_ref, wr_ref,
                  o_ref):
  ssum = jnp.concatenate([sums_ref[0], sums_ref[1]], axis=1)
  cnt = (cnt_ref[0, :, 0:1] + cnt_ref[1, :, 0:1]) * 0.5
  mean = ssum / jnp.maximum(cnt, 1.0)
  o = (jnp.dot(mean, wl_ref[...], preferred_element_type=jnp.float32)
       + bl_ref[...]
       + jnp.dot(x_ref[...], wr_ref[...], preferred_element_type=jnp.float32))
  if relu:
    o = jnp.maximum(o, 0.0)
  o_ref[...] = o


def _tc_combine(sums, cnt, x, wl_t, bl, wr_t, relu):
  B = 1000
  grid = N // B
  return pl.pallas_call(
      functools.partial(_combine_body, relu),
      grid=(grid,),
      in_specs=[
          pl.BlockSpec((NC, B, HD), lambda i: (0, i, 0)),
          pl.BlockSpec((NC, B, 16), lambda i: (0, i, 0)),
          pl.BlockSpec((B, D), lambda i: (i, 0)),
          pl.BlockSpec((D, D), lambda i: (0, 0)),
          pl.BlockSpec((1, D), lambda i: (0, 0)),
          pl.BlockSpec((D, D), lambda i: (0, 0)),
      ],
      out_specs=pl.BlockSpec((B, D), lambda i: (i, 0)),
      out_shape=jax.ShapeDtypeStruct((N, D), jnp.float32),
  )(sums, cnt, x, wl_t, bl, wr_t)


def kernel(x, edge_index, W_l1, b_l1, W_r1, W_l2, b_l2, W_r2):
  src = edge_index[0].astype(jnp.int32).reshape(NCHUNK, CHUNK)
  dst = edge_index[1].astype(jnp.int32).reshape(NCHUNK, CHUNK)

  zrows = jnp.zeros((STAGE_ROWS, HD), jnp.float32)
  zcnt = jnp.zeros((STAGE_ROWS, 16), jnp.float32)
  ones = jnp.ones((CHUNK, 16), jnp.float32)

  sums1, cnt = _sc_aggregate_counts(x.reshape(2 * N, HD), src, dst,
                                    zrows, zcnt, ones)
  sums1 = sums1.reshape(NC, PAD_N, HD)
  cnt = cnt.reshape(NC, PAD_N, 16)

  h = _tc_combine(sums1, cnt, x, W_l1.T, b_l1.reshape(1, D), W_r1.T,
                  relu=True)

  sums2 = _sc_aggregate_plain(h.reshape(2 * N, HD), src, dst, zrows)
  sums2 = sums2.reshape(NC, PAD_N, HD)
  out = _tc_combine(sums2, cnt, h, W_l2.T, b_l2.reshape(1, D), W_r2.T,
                    relu=False)
  return out
